# baseline (speedup 1.0000x reference)
import functools

import jax
import jax.numpy as jnp
from jax import lax
from jax.experimental import pallas as pl
from jax.experimental.pallas import tpu as pltpu
from jax.experimental.pallas import tpu_sc as plsc

N = 10000
E = 320000
D = 128
NC = 2
NS = 16
NW = NC * NS
EPT = E // NW
K = 80
NCHUNK = EPT // K
NP = 10240
RPS = NP // NS
ZR = 16

_EPS = 1e-16
F32 = jnp.float32


def _node_body(x_ref, cx_ref, w_ref, a_ref, hp_ref, sd_ref):
    h = jnp.concatenate([x_ref[...], cx_ref[...]], axis=-1)
    hp = jnp.dot(h, w_ref[...], preferred_element_type=F32)
    hp_ref[...] = hp
    sd_ref[...] = jnp.dot(hp, a_ref[...], preferred_element_type=F32)


def _node_call(x, cx, w, a):
    blk = 2000
    return pl.pallas_call(
        _node_body,
        grid=(N // blk,),
        in_specs=[
            pl.BlockSpec((blk, 64), lambda i: (i, 0)),
            pl.BlockSpec((blk, 64), lambda i: (i, 0)),
            pl.BlockSpec((D, D), lambda i: (0, 0)),
            pl.BlockSpec((D, 8), lambda i: (0, 0)),
        ],
        out_specs=[
            pl.BlockSpec((blk, D), lambda i: (i, 0)),
            pl.BlockSpec((blk, 8), lambda i: (i, 0)),
        ],
        out_shape=[
            jax.ShapeDtypeStruct((N, D), F32),
            jax.ShapeDtypeStruct((N, 8), F32),
        ],
    )(x, cx, w, a)


def _el_body(ea_ref, we0_ref, we1_ref, ae0_ref, ae1_ref, el_ref):
    w0 = jnp.dot(we0_ref[...], ae0_ref[...], preferred_element_type=F32)
    w1 = jnp.dot(we1_ref[...], ae1_ref[...], preferred_element_type=F32)
    el_ref[...] = jnp.dot(ea_ref[...], w0 + w1, preferred_element_type=F32)


def _el_call(ea, we0, we1, ae0, ae1):
    blk = 2000
    return pl.pallas_call(
        _el_body,
        grid=(E // blk,),
        in_specs=[
            pl.BlockSpec((blk, 16), lambda i: (i, 0)),
            pl.BlockSpec((16, D), lambda i: (0, 0)),
            pl.BlockSpec((16, D), lambda i: (0, 0)),
            pl.BlockSpec((D, 8), lambda i: (0, 0)),
            pl.BlockSpec((D, 8), lambda i: (0, 0)),
        ],
        out_specs=pl.BlockSpec((blk, 8), lambda i: (i, 0)),
        out_shape=jax.ShapeDtypeStruct((E, 8), F32),
    )(ea, we0, we1, ae0, ae1)


def _comb_body(op_ref, dp_ref, b_ref, w_ref, a_ref, hp_ref, sd_ref):
    o = op_ref[0] + op_ref[1]
    den = dp_ref[0, :, 0] + dp_ref[1, :, 0]
    h = o / (den[:, None] + _EPS) + b_ref[...]
    h = jnp.maximum(h, 0.0)
    hp = jnp.dot(h, w_ref[...], preferred_element_type=F32)
    hp_ref[...] = hp
    sd_ref[...] = jnp.dot(hp, a_ref[...], preferred_element_type=F32)


def _comb_call(op, dp, b, w, a):
    blk = 2000
    return pl.pallas_call(
        _comb_body,
        grid=(N // blk,),
        in_specs=[
            pl.BlockSpec((NC, blk, D), lambda i: (0, i, 0)),
            pl.BlockSpec((NC, blk, 1), lambda i: (0, i, 0)),
            pl.BlockSpec((1, D), lambda i: (0, 0)),
            pl.BlockSpec((D, D), lambda i: (0, 0)),
            pl.BlockSpec((D, 8), lambda i: (0, 0)),
        ],
        out_specs=[
            pl.BlockSpec((blk, D), lambda i: (i, 0)),
            pl.BlockSpec((blk, 8), lambda i: (i, 0)),
        ],
        out_shape=[
            jax.ShapeDtypeStruct((N, D), F32),
            jax.ShapeDtypeStruct((N, 8), F32),
        ],
    )(op, dp.reshape(NC, NP, 1), b.reshape(1, D), w, a)


def _final_body(op_ref, dp_ref, b_ref, wl_ref, bl_ref, y_ref):
    o = op_ref[0] + op_ref[1]
    den = dp_ref[0, :, 0] + dp_ref[1, :, 0]
    h = o / (den[:, None] + _EPS) + b_ref[...]
    y_ref[...] = jnp.dot(h, wl_ref[...], preferred_element_type=F32) + bl_ref[...]


def _final_call(op, dp, b, wl, bl):
    blk = 2000
    return pl.pallas_call(
        _final_body,
        grid=(N // blk,),
        in_specs=[
            pl.BlockSpec((NC, blk, D), lambda i: (0, i, 0)),
            pl.BlockSpec((NC, blk, 1), lambda i: (0, i, 0)),
            pl.BlockSpec((1, D), lambda i: (0, 0)),
            pl.BlockSpec((D, D), lambda i: (0, 0)),
            pl.BlockSpec((1, D), lambda i: (0, 0)),
        ],
        out_specs=pl.BlockSpec((blk, D), lambda i: (i, 0)),
        out_shape=jax.ShapeDtypeStruct((N, D), F32),
    )(op, dp.reshape(NC, NP, 1), b.reshape(1, D), wl, bl.reshape(1, D))


_MESH = plsc.VectorSubcoreMesh(core_axis_name="c", subcore_axis_name="s")

_DNUMS = lax.GatherDimensionNumbers(
    offset_dims=(), collapsed_slice_dims=(0,), start_index_map=(0,))


def _splat(vec16, e):
    idx = jnp.full((16, 1), e, jnp.int32)
    return lax.gather(vec16, idx, _DNUMS, (1,),
                      mode=lax.GatherScatterMode.PROMISE_IN_BOUNDS)


@functools.partial(
    pl.kernel,
    out_type=(
        jax.ShapeDtypeStruct((NC, NP), F32),
        jax.ShapeDtypeStruct((NC, NP, D), F32),
    ),
    mesh=_MESH,
    compiler_params=pltpu.CompilerParams(needs_layout_passes=False),
    scratch_types=[
        pltpu.VMEM((EPT,), jnp.int32),
        pltpu.VMEM((EPT,), jnp.int32),
        pltpu.VMEM((K,), F32),
        pltpu.VMEM((K,), F32),
        pltpu.VMEM((K,), F32),
        pltpu.VMEM((K,), F32),
        pltpu.VMEM((K, D), F32),
        pltpu.VMEM((ZR, D), F32),
        pltpu.VMEM((2048,), F32),
        pltpu.VMEM((16,), F32),
        pltpu.VMEM_SHARED((NP, D), F32),
        pltpu.VMEM_SHARED((NP,), F32),
    ],
)
def _gat_sc(src_hbm, dst_hbm, el_hbm, s_hbm, d_hbm, hp_hbm, g_hbm,
            den_out, out_out,
            src_v, dst_v, sv_c, dv_c, el_v, ex_c, rows_v, z_v, zd_v,
            g_v, out_sp, den_sp):
    c = lax.axis_index("c")
    sid = lax.axis_index("s")
    w = sid * NC + c
    base = w * EPT

    pltpu.sync_copy(src_hbm.at[pl.ds(base, EPT)], src_v)
    pltpu.sync_copy(dst_hbm.at[pl.ds(base, EPT)], dst_v)
    pltpu.sync_copy(g_hbm, g_v)

    zero16 = jnp.zeros((16,), F32)

    for r in range(ZR):
        for q in range(D // 16):
            z_v[r, pl.ds(q * 16, 16)] = zero16

    def _zsp(i, carry):
        pltpu.sync_copy(z_v, out_sp.at[pl.ds(sid * RPS + i * ZR, ZR)])
        return carry
    lax.fori_loop(0, RPS // ZR, _zsp, 0)

    def _zd(i, carry):
        zd_v[pl.ds(i * 16, 16)] = zero16
        return carry
    lax.fori_loop(0, 128, _zd, 0)

    @pl.when(sid == 0)
    def _():
        def _zden(i, carry):
            pltpu.sync_copy(zd_v, den_sp.at[pl.ds(i * 2048, 2048)])
            return carry
        lax.fori_loop(0, NP // 2048, _zden, 0)

    gvec = g_v[...]

    plsc.subcore_barrier()

    def _chunk(j, carry):
        jb = j * K
        pltpu.sync_copy(el_hbm.at[pl.ds(base + jb, K)], el_v)
        pltpu.sync_copy(s_hbm.at[src_v.at[pl.ds(jb, K)]], sv_c)
        pltpu.sync_copy(d_hbm.at[dst_v.at[pl.ds(jb, K)]], dv_c)
        pltpu.sync_copy(hp_hbm.at[src_v.at[pl.ds(jb, K)]], rows_v)

        def _ex16(t, carry2):
            z = (sv_c[pl.ds(t * 16, 16)] + dv_c[pl.ds(t * 16, 16)]
                 + el_v[pl.ds(t * 16, 16)])
            lg = jnp.where(z >= 0.0, z, z * 0.2)
            ex_c[pl.ds(t * 16, 16)] = jnp.exp(lg - gvec)
            return carry2
        lax.fori_loop(0, K // 16, _ex16, 0)

        pltpu.sync_copy(ex_c, den_sp.at[dst_v.at[pl.ds(jb, K)]], add=True)

        def _scale(tt, carry2):
            ex16 = ex_c[pl.ds(tt * 16, 16)]
            for e in range(16):
                spl = _splat(ex16, e)
                row = tt * 16 + e
                for q in range(D // 16):
                    rows_v[row, pl.ds(q * 16, 16)] = (
                        rows_v[row, pl.ds(q * 16, 16)] * spl)
            return carry2
        lax.fori_loop(0, K // 16, _scale, 0)

        pltpu.sync_copy(rows_v, out_sp.at[dst_v.at[pl.ds(jb, K)]],
                        add=True)
        return carry
    lax.fori_loop(0, NCHUNK, _chunk, 0)

    plsc.subcore_barrier()

    pltpu.sync_copy(out_sp.at[pl.ds(sid * RPS, RPS)],
                    out_out.at[c, pl.ds(sid * RPS, RPS)])

    @pl.when(sid == 0)
    def _():
        pltpu.sync_copy(den_sp, den_out.at[c])


def _pack_cols(v0, v1):
    z = jnp.zeros_like(v0)
    return jnp.stack([v0, v1, z, z, z, z, z, z], axis=1)


def kernel(x, cond_x, edge_index, edge_attr, t,
           W0, a_src0, a_dst0, We0, a_e0, b0,
           W1, a_src1, a_dst1, We1, a_e1, b1, Wl, bl):
    ei = edge_index.astype(jnp.int32)
    src = ei[0]
    dst = ei[1]

    A0 = _pack_cols(a_src0, a_dst0)
    A1 = _pack_cols(a_src1, a_dst1)
    AE0 = _pack_cols(a_e0, jnp.zeros_like(a_e0))
    AE1 = _pack_cols(jnp.zeros_like(a_e1), a_e1)

    hp0, sd0 = _node_call(x, cond_x, W0, A0)
    el01 = _el_call(edge_attr, We0, We1, AE0, AE1)
    el0 = el01[:, 0]
    el1 = el01[:, 1]

    s0, d0 = sd0[:, 0], sd0[:, 1]
    g0 = jnp.maximum(jnp.max(s0) + jnp.max(d0) + jnp.max(el0), 0.0)
    den0, outp0 = _gat_sc(src, dst, el0, s0, d0, hp0,
                          jnp.full((16,), g0, F32))

    hp1, sd1 = _comb_call(outp0, den0, b0, W1, A1)
    s1, d1 = sd1[:, 0], sd1[:, 1]
    g1 = jnp.maximum(jnp.max(s1) + jnp.max(d1) + jnp.max(el1), 0.0)
    den1, outp1 = _gat_sc(src, dst, el1, s1, d1, hp1,
                          jnp.full((16,), g1, F32))

    return _final_call(outp1, den1, b1, Wl, bl)


# speedup vs baseline: 1.2489x; 1.2489x over previous
import functools

import jax
import jax.numpy as jnp
from jax import lax
from jax.experimental import pallas as pl
from jax.experimental.pallas import tpu as pltpu
from jax.experimental.pallas import tpu_sc as plsc

N = 10000
E = 320000
D = 128
NC = 2
NS = 16
NW = NC * NS
EPT = E // NW
K = 80
NCHUNK = EPT // K
NP = 10240
RPS = NP // NS
ZR = 16
CK = 400
NSB = EPT // CK

_EPS = 1e-16
F32 = jnp.float32


def _node_body(x_ref, cx_ref, w_ref, a_ref, hp_ref, sd_ref):
    h = jnp.concatenate([x_ref[...], cx_ref[...]], axis=-1)
    hp = jnp.dot(h, w_ref[...], preferred_element_type=F32)
    hp_ref[...] = hp
    sd_ref[...] = jnp.dot(hp, a_ref[...], preferred_element_type=F32)


def _node_call(x, cx, w, a):
    blk = 2000
    return pl.pallas_call(
        _node_body,
        grid=(N // blk,),
        in_specs=[
            pl.BlockSpec((blk, 64), lambda i: (i, 0)),
            pl.BlockSpec((blk, 64), lambda i: (i, 0)),
            pl.BlockSpec((D, D), lambda i: (0, 0)),
            pl.BlockSpec((D, 8), lambda i: (0, 0)),
        ],
        out_specs=[
            pl.BlockSpec((blk, D), lambda i: (i, 0)),
            pl.BlockSpec((blk, 8), lambda i: (i, 0)),
        ],
        out_shape=[
            jax.ShapeDtypeStruct((N, D), F32),
            jax.ShapeDtypeStruct((N, 8), F32),
        ],
    )(x, cx, w, a)


def _el_body(ea_ref, we0_ref, we1_ref, ae0_ref, ae1_ref, el_ref):
    w0 = jnp.dot(we0_ref[...], ae0_ref[...], preferred_element_type=F32)
    w1 = jnp.dot(we1_ref[...], ae1_ref[...], preferred_element_type=F32)
    el_ref[...] = jnp.dot(ea_ref[...], w0 + w1, preferred_element_type=F32)


def _el_call(ea, we0, we1, ae0, ae1):
    blk = 2000
    return pl.pallas_call(
        _el_body,
        grid=(E // blk,),
        in_specs=[
            pl.BlockSpec((blk, 16), lambda i: (i, 0)),
            pl.BlockSpec((16, D), lambda i: (0, 0)),
            pl.BlockSpec((16, D), lambda i: (0, 0)),
            pl.BlockSpec((D, 8), lambda i: (0, 0)),
            pl.BlockSpec((D, 8), lambda i: (0, 0)),
        ],
        out_specs=pl.BlockSpec((blk, 8), lambda i: (i, 0)),
        out_shape=jax.ShapeDtypeStruct((E, 8), F32),
    )(ea, we0, we1, ae0, ae1)


def _comb_body(op_ref, dp_ref, b_ref, w_ref, a_ref, hp_ref, sd_ref):
    o = op_ref[0] + op_ref[1]
    den = dp_ref[0, :, 0] + dp_ref[1, :, 0]
    h = o / (den[:, None] + _EPS) + b_ref[...]
    h = jnp.maximum(h, 0.0)
    hp = jnp.dot(h, w_ref[...], preferred_element_type=F32)
    hp_ref[...] = hp
    sd_ref[...] = jnp.dot(hp, a_ref[...], preferred_element_type=F32)


def _comb_call(op, dp, b, w, a):
    blk = 2000
    return pl.pallas_call(
        _comb_body,
        grid=(N // blk,),
        in_specs=[
            pl.BlockSpec((NC, blk, D), lambda i: (0, i, 0)),
            pl.BlockSpec((NC, blk, 1), lambda i: (0, i, 0)),
            pl.BlockSpec((1, D), lambda i: (0, 0)),
            pl.BlockSpec((D, D), lambda i: (0, 0)),
            pl.BlockSpec((D, 8), lambda i: (0, 0)),
        ],
        out_specs=[
            pl.BlockSpec((blk, D), lambda i: (i, 0)),
            pl.BlockSpec((blk, 8), lambda i: (i, 0)),
        ],
        out_shape=[
            jax.ShapeDtypeStruct((N, D), F32),
            jax.ShapeDtypeStruct((N, 8), F32),
        ],
    )(op, dp.reshape(NC, NP, 1), b.reshape(1, D), w, a)


def _final_body(op_ref, dp_ref, b_ref, wl_ref, bl_ref, y_ref):
    o = op_ref[0] + op_ref[1]
    den = dp_ref[0, :, 0] + dp_ref[1, :, 0]
    h = o / (den[:, None] + _EPS) + b_ref[...]
    y_ref[...] = jnp.dot(h, wl_ref[...], preferred_element_type=F32) + bl_ref[...]


def _final_call(op, dp, b, wl, bl):
    blk = 2000
    return pl.pallas_call(
        _final_body,
        grid=(N // blk,),
        in_specs=[
            pl.BlockSpec((NC, blk, D), lambda i: (0, i, 0)),
            pl.BlockSpec((NC, blk, 1), lambda i: (0, i, 0)),
            pl.BlockSpec((1, D), lambda i: (0, 0)),
            pl.BlockSpec((D, D), lambda i: (0, 0)),
            pl.BlockSpec((1, D), lambda i: (0, 0)),
        ],
        out_specs=pl.BlockSpec((blk, D), lambda i: (i, 0)),
        out_shape=jax.ShapeDtypeStruct((N, D), F32),
    )(op, dp.reshape(NC, NP, 1), b.reshape(1, D), wl, bl.reshape(1, D))


_MESH = plsc.VectorSubcoreMesh(core_axis_name="c", subcore_axis_name="s")

_DNUMS = lax.GatherDimensionNumbers(
    offset_dims=(), collapsed_slice_dims=(0,), start_index_map=(0,))


def _splat(vec16, e):
    idx = jnp.full((16, 1), e, jnp.int32)
    return lax.gather(vec16, idx, _DNUMS, (1,),
                      mode=lax.GatherScatterMode.PROMISE_IN_BOUNDS)


@functools.partial(
    pl.kernel,
    out_type=(
        jax.ShapeDtypeStruct((NC, NP), F32),
        jax.ShapeDtypeStruct((NC, NP, D), F32),
    ),
    mesh=_MESH,
    compiler_params=pltpu.CompilerParams(needs_layout_passes=False),
    scratch_types=[
        pltpu.VMEM((EPT,), jnp.int32),
        pltpu.VMEM((EPT,), jnp.int32),
        pltpu.VMEM((CK,), F32),
        pltpu.VMEM((CK,), F32),
        pltpu.VMEM((CK,), F32),
        pltpu.VMEM((EPT,), F32),
        pltpu.VMEM((K, D), F32),
        pltpu.VMEM((ZR, D), F32),
        pltpu.VMEM((2048,), F32),
        pltpu.VMEM((16,), F32),
        pltpu.VMEM_SHARED((NP, D), F32),
        pltpu.VMEM_SHARED((NP,), F32),
    ],
)
def _gat_sc(src_hbm, dst_hbm, el_hbm, s_hbm, d_hbm, hp_hbm, g_hbm,
            den_out, out_out,
            src_v, dst_v, sv_c, dv_c, el_v, ex_c, rows_v, z_v, zd_v,
            g_v, out_sp, den_sp):
    c = lax.axis_index("c")
    sid = lax.axis_index("s")
    w = sid * NC + c
    base = w * EPT

    pltpu.sync_copy(src_hbm.at[pl.ds(base, EPT)], src_v)
    pltpu.sync_copy(dst_hbm.at[pl.ds(base, EPT)], dst_v)
    pltpu.sync_copy(g_hbm, g_v)

    zero16 = jnp.zeros((16,), F32)

    for r in range(ZR):
        for q in range(D // 16):
            z_v[r, pl.ds(q * 16, 16)] = zero16

    def _zsp(i, carry):
        pltpu.sync_copy(z_v, out_sp.at[pl.ds(sid * RPS + i * ZR, ZR)])
        return carry
    lax.fori_loop(0, RPS // ZR, _zsp, 0)

    def _zd(i, carry):
        zd_v[pl.ds(i * 16, 16)] = zero16
        return carry
    lax.fori_loop(0, 128, _zd, 0)

    @pl.when(sid == 0)
    def _():
        def _zden(i, carry):
            pltpu.sync_copy(zd_v, den_sp.at[pl.ds(i * 2048, 2048)])
            return carry
        lax.fori_loop(0, NP // 2048, _zden, 0)

    gvec = g_v[...]

    plsc.subcore_barrier()

    def _sblk(j, carry):
        jb = j * CK
        pltpu.sync_copy(el_hbm.at[pl.ds(base + jb, CK)], el_v)
        pltpu.sync_copy(s_hbm.at[src_v.at[pl.ds(jb, CK)]], sv_c)
        pltpu.sync_copy(d_hbm.at[dst_v.at[pl.ds(jb, CK)]], dv_c)

        def _ex16(t, carry2):
            z = (sv_c[pl.ds(t * 16, 16)] + dv_c[pl.ds(t * 16, 16)]
                 + el_v[pl.ds(t * 16, 16)])
            lg = jnp.where(z >= 0.0, z, z * 0.2)
            ex_c[pl.ds(jb + t * 16, 16)] = jnp.exp(lg - gvec)
            return carry2
        lax.fori_loop(0, CK // 16, _ex16, 0)

        pltpu.sync_copy(ex_c.at[pl.ds(jb, CK)],
                        den_sp.at[dst_v.at[pl.ds(jb, CK)]], add=True)
        return carry
    lax.fori_loop(0, NSB, _sblk, 0)

    def _chunk(j, carry):
        jb = j * K
        pltpu.sync_copy(hp_hbm.at[src_v.at[pl.ds(jb, K)]], rows_v)

        def _scale(tt, carry2):
            ex16 = ex_c[pl.ds(jb + tt * 16, 16)]
            for e in range(16):
                spl = _splat(ex16, e)
                row = tt * 16 + e
                for q in range(D // 16):
                    rows_v[row, pl.ds(q * 16, 16)] = (
                        rows_v[row, pl.ds(q * 16, 16)] * spl)
            return carry2
        lax.fori_loop(0, K // 16, _scale, 0)

        pltpu.sync_copy(rows_v, out_sp.at[dst_v.at[pl.ds(jb, K)]],
                        add=True)
        return carry
    lax.fori_loop(0, NCHUNK, _chunk, 0)

    plsc.subcore_barrier()

    pltpu.sync_copy(out_sp.at[pl.ds(sid * RPS, RPS)],
                    out_out.at[c, pl.ds(sid * RPS, RPS)])

    @pl.when(sid == 0)
    def _():
        pltpu.sync_copy(den_sp, den_out.at[c])


def _pack_cols(v0, v1):
    z = jnp.zeros_like(v0)
    return jnp.stack([v0, v1, z, z, z, z, z, z], axis=1)


def kernel(x, cond_x, edge_index, edge_attr, t,
           W0, a_src0, a_dst0, We0, a_e0, b0,
           W1, a_src1, a_dst1, We1, a_e1, b1, Wl, bl):
    ei = edge_index.astype(jnp.int32)
    src = ei[0]
    dst = ei[1]

    A0 = _pack_cols(a_src0, a_dst0)
    A1 = _pack_cols(a_src1, a_dst1)
    AE0 = _pack_cols(a_e0, jnp.zeros_like(a_e0))
    AE1 = _pack_cols(jnp.zeros_like(a_e1), a_e1)

    hp0, sd0 = _node_call(x, cond_x, W0, A0)
    el01 = _el_call(edge_attr, We0, We1, AE0, AE1)
    el0 = el01[:, 0]
    el1 = el01[:, 1]

    s0, d0 = sd0[:, 0], sd0[:, 1]
    g0 = jnp.maximum(jnp.max(s0) + jnp.max(d0) + jnp.max(el0), 0.0)
    den0, outp0 = _gat_sc(src, dst, el0, s0, d0, hp0,
                          jnp.full((16,), g0, F32))

    hp1, sd1 = _comb_call(outp0, den0, b0, W1, A1)
    s1, d1 = sd1[:, 0], sd1[:, 1]
    g1 = jnp.maximum(jnp.max(s1) + jnp.max(d1) + jnp.max(el1), 0.0)
    den1, outp1 = _gat_sc(src, dst, el1, s1, d1, hp1,
                          jnp.full((16,), g1, F32))

    return _final_call(outp1, den1, b1, Wl, bl)


# speedup vs baseline: 1.2980x; 1.0393x over previous
import functools

import jax
import jax.numpy as jnp
from jax import lax
from jax.experimental import pallas as pl
from jax.experimental.pallas import tpu as pltpu
from jax.experimental.pallas import tpu_sc as plsc

N = 10000
E = 320000
D = 128
NC = 2
NS = 16
NW = NC * NS
EPT = E // NW
K = 80
NCHUNK = EPT // K
NP = 10240
RPS = NP // NS
ZR = 16
CK = 400
NSB = EPT // CK
K2 = 40
NCHUNK2 = EPT // K2

_EPS = 1e-16
F32 = jnp.float32


def _node_body(x_ref, cx_ref, w_ref, a_ref, hp_ref, sd_ref):
    h = jnp.concatenate([x_ref[...], cx_ref[...]], axis=-1)
    hp = jnp.dot(h, w_ref[...], preferred_element_type=F32)
    hp_ref[...] = hp
    sd_ref[...] = jnp.dot(hp, a_ref[...], preferred_element_type=F32)


def _node_call(x, cx, w, a):
    blk = 2000
    return pl.pallas_call(
        _node_body,
        grid=(N // blk,),
        in_specs=[
            pl.BlockSpec((blk, 64), lambda i: (i, 0)),
            pl.BlockSpec((blk, 64), lambda i: (i, 0)),
            pl.BlockSpec((D, D), lambda i: (0, 0)),
            pl.BlockSpec((D, 8), lambda i: (0, 0)),
        ],
        out_specs=[
            pl.BlockSpec((blk, D), lambda i: (i, 0)),
            pl.BlockSpec((blk, 8), lambda i: (i, 0)),
        ],
        out_shape=[
            jax.ShapeDtypeStruct((N, D), F32),
            jax.ShapeDtypeStruct((N, 8), F32),
        ],
    )(x, cx, w, a)


def _el_body(ea_ref, we0_ref, we1_ref, ae0_ref, ae1_ref, el_ref):
    w0 = jnp.dot(we0_ref[...], ae0_ref[...], preferred_element_type=F32)
    w1 = jnp.dot(we1_ref[...], ae1_ref[...], preferred_element_type=F32)
    el_ref[...] = jnp.dot(ea_ref[...], w0 + w1, preferred_element_type=F32)


def _el_call(ea, we0, we1, ae0, ae1):
    blk = 2000
    return pl.pallas_call(
        _el_body,
        grid=(E // blk,),
        in_specs=[
            pl.BlockSpec((blk, 16), lambda i: (i, 0)),
            pl.BlockSpec((16, D), lambda i: (0, 0)),
            pl.BlockSpec((16, D), lambda i: (0, 0)),
            pl.BlockSpec((D, 8), lambda i: (0, 0)),
            pl.BlockSpec((D, 8), lambda i: (0, 0)),
        ],
        out_specs=pl.BlockSpec((blk, 8), lambda i: (i, 0)),
        out_shape=jax.ShapeDtypeStruct((E, 8), F32),
    )(ea, we0, we1, ae0, ae1)


def _comb_body(op_ref, dp_ref, b_ref, w_ref, a_ref, hp_ref, sd_ref):
    o = op_ref[0] + op_ref[1]
    den = dp_ref[0, :, 0] + dp_ref[1, :, 0]
    h = o / (den[:, None] + _EPS) + b_ref[...]
    h = jnp.maximum(h, 0.0)
    hp = jnp.dot(h, w_ref[...], preferred_element_type=F32)
    hp_ref[...] = hp
    sd_ref[...] = jnp.dot(hp, a_ref[...], preferred_element_type=F32)


def _comb_call(op, dp, b, w, a):
    blk = 2000
    return pl.pallas_call(
        _comb_body,
        grid=(N // blk,),
        in_specs=[
            pl.BlockSpec((NC, blk, D), lambda i: (0, i, 0)),
            pl.BlockSpec((NC, blk, 1), lambda i: (0, i, 0)),
            pl.BlockSpec((1, D), lambda i: (0, 0)),
            pl.BlockSpec((D, D), lambda i: (0, 0)),
            pl.BlockSpec((D, 8), lambda i: (0, 0)),
        ],
        out_specs=[
            pl.BlockSpec((blk, D), lambda i: (i, 0)),
            pl.BlockSpec((blk, 8), lambda i: (i, 0)),
        ],
        out_shape=[
            jax.ShapeDtypeStruct((N, D), F32),
            jax.ShapeDtypeStruct((N, 8), F32),
        ],
    )(op, dp.reshape(NC, NP, 1), b.reshape(1, D), w, a)


def _final_body(op_ref, dp_ref, b_ref, wl_ref, bl_ref, y_ref):
    o = op_ref[0] + op_ref[1]
    den = dp_ref[0, :, 0] + dp_ref[1, :, 0]
    h = o / (den[:, None] + _EPS) + b_ref[...]
    y_ref[...] = jnp.dot(h, wl_ref[...], preferred_element_type=F32) + bl_ref[...]


def _final_call(op, dp, b, wl, bl):
    blk = 2000
    return pl.pallas_call(
        _final_body,
        grid=(N // blk,),
        in_specs=[
            pl.BlockSpec((NC, blk, D), lambda i: (0, i, 0)),
            pl.BlockSpec((NC, blk, 1), lambda i: (0, i, 0)),
            pl.BlockSpec((1, D), lambda i: (0, 0)),
            pl.BlockSpec((D, D), lambda i: (0, 0)),
            pl.BlockSpec((1, D), lambda i: (0, 0)),
        ],
        out_specs=pl.BlockSpec((blk, D), lambda i: (i, 0)),
        out_shape=jax.ShapeDtypeStruct((N, D), F32),
    )(op, dp.reshape(NC, NP, 1), b.reshape(1, D), wl, bl.reshape(1, D))


_MESH = plsc.VectorSubcoreMesh(core_axis_name="c", subcore_axis_name="s")

_DNUMS = lax.GatherDimensionNumbers(
    offset_dims=(), collapsed_slice_dims=(0,), start_index_map=(0,))


def _splat(vec16, e):
    idx = jnp.full((16, 1), e, jnp.int32)
    return lax.gather(vec16, idx, _DNUMS, (1,),
                      mode=lax.GatherScatterMode.PROMISE_IN_BOUNDS)


@functools.partial(
    pl.kernel,
    out_type=(
        jax.ShapeDtypeStruct((NC, NP), F32),
        jax.ShapeDtypeStruct((NC, NP, D), F32),
    ),
    mesh=_MESH,
    compiler_params=pltpu.CompilerParams(needs_layout_passes=False),
    scratch_types=[
        pltpu.VMEM((EPT,), jnp.int32),
        pltpu.VMEM((EPT,), jnp.int32),
        pltpu.VMEM((CK,), F32),
        pltpu.VMEM((CK,), F32),
        pltpu.VMEM((CK,), F32),
        pltpu.VMEM((EPT,), F32),
        pltpu.VMEM((2, K2, D), F32),
        pltpu.VMEM((ZR, D), F32),
        pltpu.VMEM((2048,), F32),
        pltpu.VMEM((16,), F32),
        pltpu.SemaphoreType.DMA,
        pltpu.SemaphoreType.DMA,
        pltpu.VMEM_SHARED((NP, D), F32),
        pltpu.VMEM_SHARED((NP,), F32),
    ],
)
def _gat_sc(src_hbm, dst_hbm, el_hbm, s_hbm, d_hbm, hp_hbm, g_hbm,
            den_out, out_out,
            src_v, dst_v, sv_c, dv_c, el_v, ex_c, rows_v, z_v, zd_v,
            g_v, sem0, sem1, out_sp, den_sp):
    c = lax.axis_index("c")
    sid = lax.axis_index("s")
    w = sid * NC + c
    base = w * EPT

    pltpu.sync_copy(src_hbm.at[pl.ds(base, EPT)], src_v)
    pltpu.sync_copy(dst_hbm.at[pl.ds(base, EPT)], dst_v)
    pltpu.sync_copy(g_hbm, g_v)

    zero16 = jnp.zeros((16,), F32)

    for r in range(ZR):
        for q in range(D // 16):
            z_v[r, pl.ds(q * 16, 16)] = zero16

    def _zsp(i, carry):
        pltpu.sync_copy(z_v, out_sp.at[pl.ds(sid * RPS + i * ZR, ZR)])
        return carry
    lax.fori_loop(0, RPS // ZR, _zsp, 0)

    def _zd(i, carry):
        zd_v[pl.ds(i * 16, 16)] = zero16
        return carry
    lax.fori_loop(0, 128, _zd, 0)

    @pl.when(sid == 0)
    def _():
        def _zden(i, carry):
            pltpu.sync_copy(zd_v, den_sp.at[pl.ds(i * 2048, 2048)])
            return carry
        lax.fori_loop(0, NP // 2048, _zden, 0)

    gvec = g_v[...]

    plsc.subcore_barrier()

    def _sblk(j, carry):
        jb = j * CK
        pltpu.sync_copy(el_hbm.at[pl.ds(base + jb, CK)], el_v)
        pltpu.sync_copy(s_hbm.at[src_v.at[pl.ds(jb, CK)]], sv_c)
        pltpu.sync_copy(d_hbm.at[dst_v.at[pl.ds(jb, CK)]], dv_c)

        def _ex16(t, carry2):
            z = (sv_c[pl.ds(t * 16, 16)] + dv_c[pl.ds(t * 16, 16)]
                 + el_v[pl.ds(t * 16, 16)])
            lg = jnp.where(z >= 0.0, z, z * 0.2)
            ex_c[pl.ds(jb + t * 16, 16)] = jnp.exp(lg - gvec)
            return carry2
        lax.fori_loop(0, CK // 16, _ex16, 0)

        pltpu.sync_copy(ex_c.at[pl.ds(jb, CK)],
                        den_sp.at[dst_v.at[pl.ds(jb, CK)]], add=True)
        return carry
    lax.fori_loop(0, NSB, _sblk, 0)

    def _gath(j, buf, sem):
        pltpu.async_copy(hp_hbm.at[src_v.at[pl.ds(j * K2, K2)]],
                         rows_v.at[buf], sem)

    def _gwait(j, buf, sem):
        pltpu.make_async_copy(hp_hbm.at[src_v.at[pl.ds(j * K2, K2)]],
                              rows_v.at[buf], sem).wait()

    def _ss(buf, j):
        jb = j * K2
        for t in range(2):
            ex16 = ex_c[pl.ds(jb + t * 16, 16)]
            for e in range(16):
                spl = _splat(ex16, e)
                row = t * 16 + e
                for q in range(D // 16):
                    rows_v[buf, row, pl.ds(q * 16, 16)] = (
                        rows_v[buf, row, pl.ds(q * 16, 16)] * spl)
        exw = ex_c[pl.ds(jb + 24, 16)]
        for e in range(8, 16):
            spl = _splat(exw, e)
            row = 24 + e
            for q in range(D // 16):
                rows_v[buf, row, pl.ds(q * 16, 16)] = (
                    rows_v[buf, row, pl.ds(q * 16, 16)] * spl)
        pltpu.sync_copy(rows_v.at[buf],
                        out_sp.at[dst_v.at[pl.ds(jb, K2)]], add=True)

    _gath(0, 0, sem0)

    def _pair(p, carry):
        j0 = 2 * p
        _gwait(j0, 0, sem0)
        _gath(j0 + 1, 1, sem1)
        _ss(0, j0)
        _gwait(j0 + 1, 1, sem1)
        _gath(j0 + 2, 0, sem0)
        _ss(1, j0 + 1)
        return carry
    lax.fori_loop(0, NCHUNK2 // 2 - 1, _pair, 0)

    _gwait(NCHUNK2 - 2, 0, sem0)
    _gath(NCHUNK2 - 1, 1, sem1)
    _ss(0, NCHUNK2 - 2)
    _gwait(NCHUNK2 - 1, 1, sem1)
    _ss(1, NCHUNK2 - 1)

    plsc.subcore_barrier()

    pltpu.sync_copy(out_sp.at[pl.ds(sid * RPS, RPS)],
                    out_out.at[c, pl.ds(sid * RPS, RPS)])

    @pl.when(sid == 0)
    def _():
        pltpu.sync_copy(den_sp, den_out.at[c])


def _pack_cols(v0, v1):
    z = jnp.zeros_like(v0)
    return jnp.stack([v0, v1, z, z, z, z, z, z], axis=1)


def kernel(x, cond_x, edge_index, edge_attr, t,
           W0, a_src0, a_dst0, We0, a_e0, b0,
           W1, a_src1, a_dst1, We1, a_e1, b1, Wl, bl):
    ei = edge_index.astype(jnp.int32)
    src = ei[0]
    dst = ei[1]

    A0 = _pack_cols(a_src0, a_dst0)
    A1 = _pack_cols(a_src1, a_dst1)
    AE0 = _pack_cols(a_e0, jnp.zeros_like(a_e0))
    AE1 = _pack_cols(jnp.zeros_like(a_e1), a_e1)

    hp0, sd0 = _node_call(x, cond_x, W0, A0)
    el01 = _el_call(edge_attr, We0, We1, AE0, AE1)
    el0 = el01[:, 0]
    el1 = el01[:, 1]

    s0, d0 = sd0[:, 0], sd0[:, 1]
    g0 = jnp.maximum(jnp.max(s0) + jnp.max(d0) + jnp.max(el0), 0.0)
    den0, outp0 = _gat_sc(src, dst, el0, s0, d0, hp0,
                          jnp.full((16,), g0, F32))

    hp1, sd1 = _comb_call(outp0, den0, b0, W1, A1)
    s1, d1 = sd1[:, 0], sd1[:, 1]
    g1 = jnp.maximum(jnp.max(s1) + jnp.max(d1) + jnp.max(el1), 0.0)
    den1, outp1 = _gat_sc(src, dst, el1, s1, d1, hp1,
                          jnp.full((16,), g1, F32))

    return _final_call(outp1, den1, b1, Wl, bl)


# speedup vs baseline: 1.3232x; 1.0194x over previous
import functools

import jax
import jax.numpy as jnp
from jax import lax
from jax.experimental import pallas as pl
from jax.experimental.pallas import tpu as pltpu
from jax.experimental.pallas import tpu_sc as plsc

N = 10000
E = 320000
D = 128
NC = 2
NS = 16
NW = NC * NS
EPT = E // NW
K = 80
NCHUNK = EPT // K
NP = 10240
RPS = NP // NS
ZR = 16
CK = 400
NSB = EPT // CK
K2 = 40
NCHUNK2 = EPT // K2

_EPS = 1e-16
F32 = jnp.float32


def _node_body(x_ref, cx_ref, w_ref, a_ref, hp_ref, sd_ref, m_ref):
    h = jnp.concatenate([x_ref[...], cx_ref[...]], axis=-1)
    hp = jnp.dot(h, w_ref[...], preferred_element_type=F32)
    hp_ref[...] = hp
    sd = jnp.dot(hp, a_ref[...], preferred_element_type=F32)
    sd_ref[...] = sd
    cur = jnp.max(sd, axis=0, keepdims=True)

    @pl.when(pl.program_id(0) == 0)
    def _():
        m_ref[...] = cur

    @pl.when(pl.program_id(0) != 0)
    def _():
        m_ref[...] = jnp.maximum(m_ref[...], cur)


def _node_call(x, cx, w, a):
    blk = 2000
    return pl.pallas_call(
        _node_body,
        grid=(N // blk,),
        in_specs=[
            pl.BlockSpec((blk, 64), lambda i: (i, 0)),
            pl.BlockSpec((blk, 64), lambda i: (i, 0)),
            pl.BlockSpec((D, D), lambda i: (0, 0)),
            pl.BlockSpec((D, 8), lambda i: (0, 0)),
        ],
        out_specs=[
            pl.BlockSpec((blk, D), lambda i: (i, 0)),
            pl.BlockSpec((blk, 8), lambda i: (i, 0)),
            pl.BlockSpec((1, 8), lambda i: (0, 0)),
        ],
        out_shape=[
            jax.ShapeDtypeStruct((N, D), F32),
            jax.ShapeDtypeStruct((N, 8), F32),
            jax.ShapeDtypeStruct((1, 8), F32),
        ],
    )(x, cx, w, a)


def _el_body(ea_ref, we0_ref, we1_ref, ae0_ref, ae1_ref,
             el0_ref, el1_ref, m_ref):
    w0 = jnp.dot(we0_ref[...], ae0_ref[...], preferred_element_type=F32)
    w1 = jnp.dot(we1_ref[...], ae1_ref[...], preferred_element_type=F32)
    elv = jnp.dot(ea_ref[...], w0 + w1, preferred_element_type=F32)
    el0_ref[...] = elv[:, 0:1]
    el1_ref[...] = elv[:, 1:2]
    cur = jnp.max(elv, axis=0, keepdims=True)

    @pl.when(pl.program_id(0) == 0)
    def _():
        m_ref[...] = cur

    @pl.when(pl.program_id(0) != 0)
    def _():
        m_ref[...] = jnp.maximum(m_ref[...], cur)


def _el_call(ea, we0, we1, ae0, ae1):
    blk = 2000
    return pl.pallas_call(
        _el_body,
        grid=(E // blk,),
        in_specs=[
            pl.BlockSpec((blk, 16), lambda i: (i, 0)),
            pl.BlockSpec((16, D), lambda i: (0, 0)),
            pl.BlockSpec((16, D), lambda i: (0, 0)),
            pl.BlockSpec((D, 8), lambda i: (0, 0)),
            pl.BlockSpec((D, 8), lambda i: (0, 0)),
        ],
        out_specs=[
            pl.BlockSpec((blk, 1), lambda i: (i, 0)),
            pl.BlockSpec((blk, 1), lambda i: (i, 0)),
            pl.BlockSpec((1, 8), lambda i: (0, 0)),
        ],
        out_shape=[
            jax.ShapeDtypeStruct((E, 1), F32),
            jax.ShapeDtypeStruct((E, 1), F32),
            jax.ShapeDtypeStruct((1, 8), F32),
        ],
    )(ea, we0, we1, ae0, ae1)


def _comb_body(op_ref, dp_ref, b_ref, w_ref, a_ref, hp_ref, sd_ref, m_ref):
    o = op_ref[0] + op_ref[1]
    den = dp_ref[0, :, 0] + dp_ref[1, :, 0]
    h = o / (den[:, None] + _EPS) + b_ref[...]
    h = jnp.maximum(h, 0.0)
    hp = jnp.dot(h, w_ref[...], preferred_element_type=F32)
    hp_ref[...] = hp
    sd = jnp.dot(hp, a_ref[...], preferred_element_type=F32)
    sd_ref[...] = sd
    cur = jnp.max(sd, axis=0, keepdims=True)

    @pl.when(pl.program_id(0) == 0)
    def _():
        m_ref[...] = cur

    @pl.when(pl.program_id(0) != 0)
    def _():
        m_ref[...] = jnp.maximum(m_ref[...], cur)


def _comb_call(op, dp, b, w, a):
    blk = 2000
    return pl.pallas_call(
        _comb_body,
        grid=(N // blk,),
        in_specs=[
            pl.BlockSpec((NC, blk, D), lambda i: (0, i, 0)),
            pl.BlockSpec((NC, blk, 1), lambda i: (0, i, 0)),
            pl.BlockSpec((1, D), lambda i: (0, 0)),
            pl.BlockSpec((D, D), lambda i: (0, 0)),
            pl.BlockSpec((D, 8), lambda i: (0, 0)),
        ],
        out_specs=[
            pl.BlockSpec((blk, D), lambda i: (i, 0)),
            pl.BlockSpec((blk, 8), lambda i: (i, 0)),
            pl.BlockSpec((1, 8), lambda i: (0, 0)),
        ],
        out_shape=[
            jax.ShapeDtypeStruct((N, D), F32),
            jax.ShapeDtypeStruct((N, 8), F32),
            jax.ShapeDtypeStruct((1, 8), F32),
        ],
    )(op, dp.reshape(NC, NP, 1), b.reshape(1, D), w, a)


def _final_body(op_ref, dp_ref, b_ref, wl_ref, bl_ref, y_ref):
    o = op_ref[0] + op_ref[1]
    den = dp_ref[0, :, 0] + dp_ref[1, :, 0]
    h = o / (den[:, None] + _EPS) + b_ref[...]
    y_ref[...] = jnp.dot(h, wl_ref[...], preferred_element_type=F32) + bl_ref[...]


def _final_call(op, dp, b, wl, bl):
    blk = 2000
    return pl.pallas_call(
        _final_body,
        grid=(N // blk,),
        in_specs=[
            pl.BlockSpec((NC, blk, D), lambda i: (0, i, 0)),
            pl.BlockSpec((NC, blk, 1), lambda i: (0, i, 0)),
            pl.BlockSpec((1, D), lambda i: (0, 0)),
            pl.BlockSpec((D, D), lambda i: (0, 0)),
            pl.BlockSpec((1, D), lambda i: (0, 0)),
        ],
        out_specs=pl.BlockSpec((blk, D), lambda i: (i, 0)),
        out_shape=jax.ShapeDtypeStruct((N, D), F32),
    )(op, dp.reshape(NC, NP, 1), b.reshape(1, D), wl, bl.reshape(1, D))


_MESH = plsc.VectorSubcoreMesh(core_axis_name="c", subcore_axis_name="s")

_DNUMS = lax.GatherDimensionNumbers(
    offset_dims=(), collapsed_slice_dims=(0,), start_index_map=(0,))


def _splat(vec16, e):
    idx = jnp.full((16, 1), e, jnp.int32)
    return lax.gather(vec16, idx, _DNUMS, (1,),
                      mode=lax.GatherScatterMode.PROMISE_IN_BOUNDS)


@functools.partial(
    pl.kernel,
    out_type=(
        jax.ShapeDtypeStruct((NC, NP), F32),
        jax.ShapeDtypeStruct((NC, NP, D), F32),
    ),
    mesh=_MESH,
    compiler_params=pltpu.CompilerParams(needs_layout_passes=False),
    scratch_types=[
        pltpu.VMEM((EPT,), jnp.int32),
        pltpu.VMEM((EPT,), jnp.int32),
        pltpu.VMEM((CK,), F32),
        pltpu.VMEM((CK,), F32),
        pltpu.VMEM((CK,), F32),
        pltpu.VMEM((EPT,), F32),
        pltpu.VMEM((2, K2, D), F32),
        pltpu.VMEM((ZR, D), F32),
        pltpu.VMEM((2048,), F32),
        pltpu.VMEM((16,), F32),
        pltpu.SemaphoreType.DMA,
        pltpu.SemaphoreType.DMA,
        pltpu.VMEM_SHARED((NP, D), F32),
        pltpu.VMEM_SHARED((NP,), F32),
    ],
)
def _gat_sc(src_hbm, dst_hbm, el_hbm, s_hbm, d_hbm, hp_hbm, g_hbm,
            den_out, out_out,
            src_v, dst_v, sv_c, dv_c, el_v, ex_c, rows_v, z_v, zd_v,
            g_v, sem0, sem1, out_sp, den_sp):
    c = lax.axis_index("c")
    sid = lax.axis_index("s")
    w = sid * NC + c
    base = w * EPT

    pltpu.sync_copy(src_hbm.at[pl.ds(base, EPT)], src_v)
    pltpu.sync_copy(dst_hbm.at[pl.ds(base, EPT)], dst_v)
    pltpu.sync_copy(g_hbm, g_v)

    zero16 = jnp.zeros((16,), F32)

    for r in range(ZR):
        for q in range(D // 16):
            z_v[r, pl.ds(q * 16, 16)] = zero16

    def _zsp(i, carry):
        pltpu.sync_copy(z_v, out_sp.at[pl.ds(sid * RPS + i * ZR, ZR)])
        return carry
    lax.fori_loop(0, RPS // ZR, _zsp, 0)

    def _zd(i, carry):
        zd_v[pl.ds(i * 16, 16)] = zero16
        return carry
    lax.fori_loop(0, 128, _zd, 0)

    @pl.when(sid == 0)
    def _():
        def _zden(i, carry):
            pltpu.sync_copy(zd_v, den_sp.at[pl.ds(i * 2048, 2048)])
            return carry
        lax.fori_loop(0, NP // 2048, _zden, 0)

    gvec = g_v[...]

    plsc.subcore_barrier()

    def _sblk(j, carry):
        jb = j * CK
        pltpu.sync_copy(el_hbm.at[pl.ds(base + jb, CK)], el_v)
        pltpu.sync_copy(s_hbm.at[src_v.at[pl.ds(jb, CK)]], sv_c)
        pltpu.sync_copy(d_hbm.at[dst_v.at[pl.ds(jb, CK)]], dv_c)

        def _ex16(t, carry2):
            z = (sv_c[pl.ds(t * 16, 16)] + dv_c[pl.ds(t * 16, 16)]
                 + el_v[pl.ds(t * 16, 16)])
            lg = jnp.where(z >= 0.0, z, z * 0.2)
            ex_c[pl.ds(jb + t * 16, 16)] = jnp.exp(lg - gvec)
            return carry2
        lax.fori_loop(0, CK // 16, _ex16, 0)

        pltpu.sync_copy(ex_c.at[pl.ds(jb, CK)],
                        den_sp.at[dst_v.at[pl.ds(jb, CK)]], add=True)
        return carry
    lax.fori_loop(0, NSB, _sblk, 0)

    def _gath(j, buf, sem):
        pltpu.async_copy(hp_hbm.at[src_v.at[pl.ds(j * K2, K2)]],
                         rows_v.at[buf], sem)

    def _gwait(j, buf, sem):
        pltpu.make_async_copy(hp_hbm.at[src_v.at[pl.ds(j * K2, K2)]],
                              rows_v.at[buf], sem).wait()

    def _ss(buf, j):
        jb = j * K2
        for t in range(2):
            ex16 = ex_c[pl.ds(jb + t * 16, 16)]
            for e in range(16):
                spl = _splat(ex16, e)
                row = t * 16 + e
                for q in range(D // 16):
                    rows_v[buf, row, pl.ds(q * 16, 16)] = (
                        rows_v[buf, row, pl.ds(q * 16, 16)] * spl)
        exw = ex_c[pl.ds(jb + 24, 16)]
        for e in range(8, 16):
            spl = _splat(exw, e)
            row = 24 + e
            for q in range(D // 16):
                rows_v[buf, row, pl.ds(q * 16, 16)] = (
                    rows_v[buf, row, pl.ds(q * 16, 16)] * spl)
        pltpu.sync_copy(rows_v.at[buf],
                        out_sp.at[dst_v.at[pl.ds(jb, K2)]], add=True)

    _gath(0, 0, sem0)

    def _pair(p, carry):
        j0 = 2 * p
        _gwait(j0, 0, sem0)
        _gath(j0 + 1, 1, sem1)
        _ss(0, j0)
        _gwait(j0 + 1, 1, sem1)
        _gath(j0 + 2, 0, sem0)
        _ss(1, j0 + 1)
        return carry
    lax.fori_loop(0, NCHUNK2 // 2 - 1, _pair, 0)

    _gwait(NCHUNK2 - 2, 0, sem0)
    _gath(NCHUNK2 - 1, 1, sem1)
    _ss(0, NCHUNK2 - 2)
    _gwait(NCHUNK2 - 1, 1, sem1)
    _ss(1, NCHUNK2 - 1)

    plsc.subcore_barrier()

    pltpu.sync_copy(out_sp.at[pl.ds(sid * RPS, RPS)],
                    out_out.at[c, pl.ds(sid * RPS, RPS)])

    @pl.when(sid == 0)
    def _():
        pltpu.sync_copy(den_sp, den_out.at[c])


def _pack_cols(v0, v1):
    z = jnp.zeros_like(v0)
    return jnp.stack([v0, v1, z, z, z, z, z, z], axis=1)


def kernel(x, cond_x, edge_index, edge_attr, t,
           W0, a_src0, a_dst0, We0, a_e0, b0,
           W1, a_src1, a_dst1, We1, a_e1, b1, Wl, bl):
    ei = edge_index.astype(jnp.int32)
    src = ei[0]
    dst = ei[1]

    A0 = _pack_cols(a_src0, a_dst0)
    A1 = _pack_cols(a_src1, a_dst1)
    AE0 = _pack_cols(a_e0, jnp.zeros_like(a_e0))
    AE1 = _pack_cols(jnp.zeros_like(a_e1), a_e1)

    hp0, sd0, m0 = _node_call(x, cond_x, W0, A0)
    el0, el1, mel = _el_call(edge_attr, We0, We1, AE0, AE1)
    el0 = el0.reshape(E)
    el1 = el1.reshape(E)

    s0, d0 = sd0[:, 0], sd0[:, 1]
    g0 = jnp.maximum(m0[0, 0] + m0[0, 1] + mel[0, 0], 0.0)
    den0, outp0 = _gat_sc(src, dst, el0, s0, d0, hp0,
                          jnp.full((16,), g0, F32))

    hp1, sd1, m1 = _comb_call(outp0, den0, b0, W1, A1)
    s1, d1 = sd1[:, 0], sd1[:, 1]
    g1 = jnp.maximum(m1[0, 0] + m1[0, 1] + mel[0, 1], 0.0)
    den1, outp1 = _gat_sc(src, dst, el1, s1, d1, hp1,
                          jnp.full((16,), g1, F32))

    return _final_call(outp1, den1, b1, Wl, bl)
